# baseline (device time: 67494 ns/iter reference)
import numpy as np
import jax
import jax.numpy as jnp
from jax import lax
from jax.experimental import pallas as pl
from jax.experimental.pallas import tpu as pltpu

N_DEV = 4
B, SQ, D = 8, 128, 512
B_LOC = B // N_DEV
HL, DH = 4, 64
HD_LOC = HL * DH
M = B_LOC * SQ
M_FULL = B * SQ


def _rope_tables():
    inv = 1.0 / (10000.0 ** (np.arange(0, DH, 2) / DH))
    pos = np.arange(SQ)[:, None] * inv[None, :]
    cos = np.repeat(np.cos(pos), 2, axis=-1).astype(np.float32)
    sin = np.repeat(np.sin(pos), 2, axis=-1).astype(np.float32)
    return np.tile(cos, (B, HL)), np.tile(sin, (B, HL))


_COS2, _SIN2 = _rope_tables()


def kernel(x, Wq, Wk, Wv, Wo):
    cos2 = jnp.asarray(_COS2)
    sin2 = jnp.asarray(_SIN2)

    def body(x_ref, wq_ref, wk_ref, wv_ref, wo_ref, cos_ref, sin_ref,
             out_ref, xfull, ctx_ref, pfull, rs_buf,
             ag_send, ag_recv, rs_send, rs_recv):
        p = lax.axis_index("i")
        left = lax.rem(p + N_DEV - 1, N_DEV)
        right = lax.rem(p + 1, N_DEV)

        barrier = pltpu.get_barrier_semaphore()
        for nbr in (left, right):
            pl.semaphore_signal(barrier, inc=1, device_id=(nbr,),
                                device_id_type=pl.DeviceIdType.MESH)
        pl.semaphore_wait(barrier, 2)

        xfull[pl.ds(p * M, M)] = x_ref[...].reshape(M, D)
        for h in range(N_DEV - 1):
            c = lax.rem(p - h + N_DEV, N_DEV)
            rdma = pltpu.make_async_remote_copy(
                src_ref=xfull.at[pl.ds(c * M, M)],
                dst_ref=xfull.at[pl.ds(c * M, M)],
                send_sem=ag_send.at[h],
                recv_sem=ag_recv.at[h],
                device_id=(right,),
                device_id_type=pl.DeviceIdType.MESH,
            )
            rdma.start()
            rdma.wait()

        xf = xfull[...]
        q2 = jnp.dot(xf, wq_ref[...], preferred_element_type=jnp.float32)
        k2 = jnp.dot(xf, wk_ref[...], preferred_element_type=jnp.float32)
        v2 = jnp.dot(xf, wv_ref[...], preferred_element_type=jnp.float32)

        cos = cos_ref[...]
        sin = sin_ref[...]
        col = lax.broadcasted_iota(jnp.int32, (M_FULL, HD_LOC), 1)
        even = (col % 2) == 0

        def rope(t):
            lshift = jnp.concatenate([t[:, 1:], t[:, :1]], axis=1)
            rshift = jnp.concatenate([t[:, -1:], t[:, :-1]], axis=1)
            t_r = jnp.where(even, -lshift, rshift)
            return t * cos + t_r * sin

        q2 = rope(q2)
        k2 = rope(k2)

        for b in range(B):
            rs = slice(b * SQ, (b + 1) * SQ)
            for hh in range(HL):
                cs = slice(hh * DH, (hh + 1) * DH)
                q = q2[rs, cs]
                k = k2[rs, cs]
                v = v2[rs, cs]
                s = lax.dot_general(q, k, (((1,), (1,)), ((), ())),
                                    preferred_element_type=jnp.float32) * 0.125
                mx = jnp.max(s, axis=-1, keepdims=True)
                w = jnp.exp(s - mx)
                w = w / jnp.sum(w, axis=-1, keepdims=True)
                ctx_ref[rs, cs] = jnp.dot(w, v,
                                          preferred_element_type=jnp.float32)

        pfull[...] = jnp.dot(ctx_ref[...], wo_ref[...],
                             preferred_element_type=jnp.float32)

        c0 = lax.rem(p + N_DEV - 1, N_DEV)
        rd0 = pltpu.make_async_remote_copy(
            src_ref=pfull.at[pl.ds(c0 * M, M)],
            dst_ref=rs_buf.at[0],
            send_sem=rs_send.at[0],
            recv_sem=rs_recv.at[0],
            device_id=(right,),
            device_id_type=pl.DeviceIdType.MESH,
        )
        rd0.start()
        rd0.wait()
        for h in range(1, N_DEV - 1):
            c = lax.rem(p + 2 * N_DEV - 1 - h, N_DEV)
            rs_buf[h - 1] = rs_buf[h - 1] + pfull[pl.ds(c * M, M)]
            rd = pltpu.make_async_remote_copy(
                src_ref=rs_buf.at[h - 1],
                dst_ref=rs_buf.at[h],
                send_sem=rs_send.at[h],
                recv_sem=rs_recv.at[h],
                device_id=(right,),
                device_id_type=pl.DeviceIdType.MESH,
            )
            rd.start()
            rd.wait()

        final = rs_buf[N_DEV - 2] + pfull[pl.ds(p * M, M)]
        out_ref[...] = final.reshape(B_LOC, SQ, D)

    return pl.pallas_call(
        body,
        out_shape=jax.ShapeDtypeStruct((B_LOC, SQ, D), jnp.float32),
        in_specs=[pl.BlockSpec(memory_space=pltpu.VMEM)] * 7,
        out_specs=pl.BlockSpec(memory_space=pltpu.VMEM),
        scratch_shapes=[
            pltpu.VMEM((M_FULL, D), jnp.float32),
            pltpu.VMEM((M_FULL, HD_LOC), jnp.float32),
            pltpu.VMEM((M_FULL, D), jnp.float32),
            pltpu.VMEM((N_DEV - 1, M, D), jnp.float32),
            pltpu.SemaphoreType.DMA((N_DEV - 1,)),
            pltpu.SemaphoreType.DMA((N_DEV - 1,)),
            pltpu.SemaphoreType.DMA((N_DEV - 1,)),
            pltpu.SemaphoreType.DMA((N_DEV - 1,)),
        ],
        compiler_params=pltpu.CompilerParams(collective_id=0),
    )(x, Wq, Wk, Wv, Wo, cos2, sin2)


# device time: 50449 ns/iter; 1.3379x vs baseline; 1.3379x over previous
import numpy as np
import jax
import jax.numpy as jnp
from jax import lax
from jax.experimental import pallas as pl
from jax.experimental.pallas import tpu as pltpu

N_DEV = 4
B, SQ, D = 8, 128, 512
B_LOC = B // N_DEV
HL, DH = 4, 64
HD_LOC = HL * DH
M = B_LOC * SQ


def _rope_tables():
    inv = 1.0 / (10000.0 ** (np.arange(0, DH, 2) / DH))
    pos = np.arange(SQ)[:, None] * inv[None, :]
    cos = np.repeat(np.cos(pos), 2, axis=-1).astype(np.float32)
    sin = np.repeat(np.sin(pos), 2, axis=-1).astype(np.float32)
    return np.tile(cos, (B_LOC, HL)), np.tile(sin, (B_LOC, HL))


_COS2, _SIN2 = _rope_tables()


def kernel(x, Wq, Wk, Wv, Wo):
    cos2 = jnp.asarray(_COS2)
    sin2 = jnp.asarray(_SIN2)

    def body(x_ref, wq_ref, wk_ref, wv_ref, wo_ref, cos_ref, sin_ref,
             out_ref, xfull, ctx_ref, pfull, rs_buf,
             ag_send, ag_recv, rs_send, rs_recv):
        p = lax.axis_index("i")
        left = lax.rem(p + N_DEV - 1, N_DEV)
        right = lax.rem(p + 1, N_DEV)

        barrier = pltpu.get_barrier_semaphore()
        for nbr in (left, right):
            pl.semaphore_signal(barrier, inc=1, device_id=(nbr,),
                                device_id_type=pl.DeviceIdType.MESH)
        pl.semaphore_wait(barrier, 2)

        cos = cos_ref[...]
        sin = sin_ref[...]
        col = lax.broadcasted_iota(jnp.int32, (M, HD_LOC), 1)
        even = (col % 2) == 0
        wq = wq_ref[...]
        wk = wk_ref[...]
        wv = wv_ref[...]
        wo = wo_ref[...]

        def rope(t):
            lshift = jnp.concatenate([t[:, 1:], t[:, :1]], axis=1)
            rshift = jnp.concatenate([t[:, -1:], t[:, :-1]], axis=1)
            t_r = jnp.where(even, -lshift, rshift)
            return t * cos + t_r * sin

        def compute_chunk(c):
            xc = xfull[pl.ds(c * M, M)]
            qc = rope(jnp.dot(xc, wq, preferred_element_type=jnp.float32))
            kc = rope(jnp.dot(xc, wk, preferred_element_type=jnp.float32))
            vc = jnp.dot(xc, wv, preferred_element_type=jnp.float32)
            for b in range(B_LOC):
                rows = slice(b * SQ, (b + 1) * SQ)
                for hh in range(HL):
                    cols = slice(hh * DH, (hh + 1) * DH)
                    q = qc[rows, cols]
                    k = kc[rows, cols]
                    v = vc[rows, cols]
                    s = lax.dot_general(
                        q, k, (((1,), (1,)), ((), ())),
                        preferred_element_type=jnp.float32) * 0.125
                    mx = jnp.max(s, axis=-1, keepdims=True)
                    w = jnp.exp(s - mx)
                    w = w / jnp.sum(w, axis=-1, keepdims=True)
                    ctx_ref[rows, cols] = jnp.dot(
                        w, v, preferred_element_type=jnp.float32)
            pfull[pl.ds(c * M, M)] = jnp.dot(
                ctx_ref[...], wo, preferred_element_type=jnp.float32)

        def ag_hop(h, c):
            r = pltpu.make_async_remote_copy(
                src_ref=xfull.at[pl.ds(c * M, M)],
                dst_ref=xfull.at[pl.ds(c * M, M)],
                send_sem=ag_send.at[h],
                recv_sem=ag_recv.at[h],
                device_id=(right,),
                device_id_type=pl.DeviceIdType.MESH,
            )
            r.start()
            return r

        def rs_hop(h, src):
            r = pltpu.make_async_remote_copy(
                src_ref=src,
                dst_ref=rs_buf.at[h],
                send_sem=rs_send.at[h],
                recv_sem=rs_recv.at[h],
                device_id=(right,),
                device_id_type=pl.DeviceIdType.MESH,
            )
            r.start()
            return r

        xfull[pl.ds(p * M, M)] = x_ref[...].reshape(M, D)
        ag_rdmas = [ag_hop(0, p)]
        rs_rdmas = []
        compute_chunk(p)

        for h in range(1, N_DEV):
            c = lax.rem(p - h + N_DEV, N_DEV)
            ag_rdmas[-1].wait_recv()
            if h < N_DEV - 1:
                ag_rdmas.append(ag_hop(h, c))
            compute_chunk(c)
            if h == 1:
                rs_rdmas.append(rs_hop(0, pfull.at[pl.ds(c * M, M)]))
            else:
                rs_rdmas[-1].wait_recv()
                rs_buf[h - 2] = rs_buf[h - 2] + pfull[pl.ds(c * M, M)]
                rs_rdmas.append(rs_hop(h - 1, rs_buf.at[h - 2]))

        rs_rdmas[-1].wait_recv()
        final = rs_buf[N_DEV - 2] + pfull[pl.ds(p * M, M)]

        for r in ag_rdmas + rs_rdmas:
            r.wait_send()

        out_ref[...] = final.reshape(B_LOC, SQ, D)

    return pl.pallas_call(
        body,
        out_shape=jax.ShapeDtypeStruct((B_LOC, SQ, D), jnp.float32),
        in_specs=[pl.BlockSpec(memory_space=pltpu.VMEM)] * 7,
        out_specs=pl.BlockSpec(memory_space=pltpu.VMEM),
        scratch_shapes=[
            pltpu.VMEM((B * SQ, D), jnp.float32),
            pltpu.VMEM((M, HD_LOC), jnp.float32),
            pltpu.VMEM((B * SQ, D), jnp.float32),
            pltpu.VMEM((N_DEV - 1, M, D), jnp.float32),
            pltpu.SemaphoreType.DMA((N_DEV - 1,)),
            pltpu.SemaphoreType.DMA((N_DEV - 1,)),
            pltpu.SemaphoreType.DMA((N_DEV - 1,)),
            pltpu.SemaphoreType.DMA((N_DEV - 1,)),
        ],
        compiler_params=pltpu.CompilerParams(collective_id=0),
    )(x, Wq, Wk, Wv, Wo, cos2, sin2)


# device time: 21076 ns/iter; 3.2024x vs baseline; 2.3937x over previous
import numpy as np
import ml_dtypes
import jax
import jax.numpy as jnp
from jax import lax
from jax.experimental import pallas as pl
from jax.experimental.pallas import tpu as pltpu

N_DEV = 4
B, SQ, D = 8, 128, 512
B_LOC = B // N_DEV
HL, DH = 4, 64
HD_LOC = HL * DH
M = B_LOC * SQ
SCALE = 0.125


def _rope_tables():
    inv = 1.0 / (10000.0 ** (np.arange(0, DH, 2) / DH))
    pos = np.arange(SQ)[:, None] * inv[None, :]
    cos = np.repeat(np.cos(pos), 2, axis=-1).astype(np.float32)
    sin = np.repeat(np.sin(pos), 2, axis=-1).astype(np.float32)
    cos_qk = np.concatenate(
        [np.tile(cos, (2 * B_LOC, HL)) * SCALE,
         np.tile(cos, (2 * B_LOC, HL))], axis=1)
    sin_qk = np.concatenate(
        [np.tile(sin, (2 * B_LOC, HL)) * SCALE,
         np.tile(sin, (2 * B_LOC, HL))], axis=1)
    r = np.arange(M) // SQ
    mask = np.where(r[:, None] == r[None, :], 0.0, -30000.0)
    packed = np.concatenate(
        [cos_qk[:M], sin_qk[:M], mask], axis=1).astype(ml_dtypes.bfloat16)
    return packed


_PACKED = _rope_tables()


def kernel(x, Wq, Wk, Wv, Wo):
    packed = jnp.asarray(_PACKED)
    wqkv_in = jnp.concatenate([Wq, Wk, Wv], axis=1)

    def body(x_ref, wqkv_ref, wo_ref, packed_ref, out_ref,
             xfull, ctx_ref, pfull, rs_buf,
             ag_send, ag_recv, rs_send, rs_recv):
        p = lax.axis_index("i")

        barrier = pltpu.get_barrier_semaphore()
        for d in (1, 2, 3):
            pl.semaphore_signal(barrier, inc=1,
                                device_id=(lax.rem(p + d, N_DEV),),
                                device_id_type=pl.DeviceIdType.MESH)
        xfull[pl.ds(p * M, M)] = x_ref[...].reshape(M, D).astype(jnp.bfloat16)
        pl.semaphore_wait(barrier, 3)

        cos = packed_ref[:, :2 * HD_LOC].astype(jnp.float32)
        sin = packed_ref[:, 2 * HD_LOC:4 * HD_LOC].astype(jnp.float32)
        msk = packed_ref[:, 4 * HD_LOC:].astype(jnp.float32)
        col = lax.broadcasted_iota(jnp.int32, (M, 2 * HD_LOC), 1)
        even = (col % 2) == 0
        wqkv = wqkv_ref[...].astype(jnp.bfloat16)
        wo = wo_ref[...].astype(jnp.bfloat16)

        def compute_chunks(cs):
            n = len(cs) * M
            xs = [xfull[pl.ds(c * M, M)] for c in cs]
            xc = xs[0] if len(cs) == 1 else jnp.concatenate(xs, axis=0)
            qkv = jnp.dot(xc, wqkv, preferred_element_type=jnp.float32)
            qk = qkv[:, :2 * HD_LOC]
            lshift = jnp.concatenate([qk[:, 1:], qk[:, :1]], axis=1)
            rshift = jnp.concatenate([qk[:, -1:], qk[:, :-1]], axis=1)
            qk_r = jnp.where(even, -lshift, rshift)
            qk = (qk * cos + qk_r * sin).astype(jnp.bfloat16)
            v = qkv[:, 2 * HD_LOC:].astype(jnp.bfloat16)
            for hh in range(HL):
                q = qk[:, hh * DH:(hh + 1) * DH]
                k = qk[:, HD_LOC + hh * DH:HD_LOC + (hh + 1) * DH]
                s = lax.dot_general(
                    q, k, (((1,), (1,)), ((), ())),
                    preferred_element_type=jnp.float32) + msk
                w = jnp.exp(s)
                norm = jnp.sum(w, axis=-1, keepdims=True)
                ctx = jnp.dot(w.astype(jnp.bfloat16),
                              v[:, hh * DH:(hh + 1) * DH],
                              preferred_element_type=jnp.float32)
                ctx_ref[:n, hh * DH:(hh + 1) * DH] = (ctx / norm).astype(
                    jnp.bfloat16)
            po = jnp.dot(ctx_ref[:n, :], wo,
                         preferred_element_type=jnp.float32)
            for i, c in enumerate(cs):
                pfull[pl.ds(c * M, M)] = po[i * M:(i + 1) * M].astype(
                    jnp.bfloat16)

        ag_rdmas = []
        for d in (1, 3, 2):
            r = pltpu.make_async_remote_copy(
                src_ref=xfull.at[pl.ds(p * M, M)],
                dst_ref=xfull.at[pl.ds(p * M, M)],
                send_sem=ag_send.at[d - 1],
                recv_sem=ag_recv.at[d - 1],
                device_id=(lax.rem(p + d, N_DEV),),
                device_id_type=pl.DeviceIdType.MESH,
            )
            r.start()
            ag_rdmas.append(r)

        def ag_wait(d):
            c = lax.rem(p - d + N_DEV, N_DEV)
            pltpu.make_async_remote_copy(
                src_ref=xfull.at[pl.ds(c * M, M)],
                dst_ref=xfull.at[pl.ds(c * M, M)],
                send_sem=ag_send.at[d - 1],
                recv_sem=ag_recv.at[d - 1],
                device_id=(lax.rem(p + d, N_DEV),),
                device_id_type=pl.DeviceIdType.MESH,
            ).wait_recv()
            return c

        def rs_push(c, d):
            r = pltpu.make_async_remote_copy(
                src_ref=pfull.at[pl.ds(c * M, M)],
                dst_ref=rs_buf.at[3 - d],
                send_sem=rs_send.at[3 - d],
                recv_sem=rs_recv.at[3 - d],
                device_id=(c,),
                device_id_type=pl.DeviceIdType.MESH,
            )
            r.start()
            return r

        compute_chunks([p])
        rs_rdmas = []
        for d in (1, 3, 2):
            c = ag_wait(d)
            compute_chunks([c])
            rs_rdmas.append(rs_push(c, d))

        def rs_wait(i):
            pltpu.make_async_remote_copy(
                src_ref=rs_buf.at[i],
                dst_ref=rs_buf.at[i],
                send_sem=rs_send.at[i],
                recv_sem=rs_recv.at[i],
                device_id=(p,),
                device_id_type=pl.DeviceIdType.MESH,
            ).wait_recv()

        rs_wait(0)
        rs_wait(2)
        part = (pfull[pl.ds(p * M, M)].astype(jnp.float32)
                + rs_buf[0].astype(jnp.float32)) + rs_buf[2].astype(
                    jnp.float32)
        rs_wait(1)
        final = part + rs_buf[1].astype(jnp.float32)

        for r in ag_rdmas + rs_rdmas:
            r.wait_send()

        out_ref[...] = final.reshape(B_LOC, SQ, D)

    return pl.pallas_call(
        body,
        out_shape=jax.ShapeDtypeStruct((B_LOC, SQ, D), jnp.float32),
        in_specs=[pl.BlockSpec(memory_space=pltpu.VMEM)] * 4,
        out_specs=pl.BlockSpec(memory_space=pltpu.VMEM),
        scratch_shapes=[
            pltpu.VMEM((B * SQ, D), jnp.bfloat16),
            pltpu.VMEM((2 * M, HD_LOC), jnp.bfloat16),
            pltpu.VMEM((B * SQ, D), jnp.bfloat16),
            pltpu.VMEM((3, M, D), jnp.bfloat16),
            pltpu.SemaphoreType.DMA((3,)),
            pltpu.SemaphoreType.DMA((3,)),
            pltpu.SemaphoreType.DMA((3,)),
            pltpu.SemaphoreType.DMA((3,)),
        ],
        compiler_params=pltpu.CompilerParams(collective_id=0),
    )(x, wqkv_in, Wo, packed)
